# baseline (speedup 1.0000x reference)
import functools

import jax
import jax.numpy as jnp
from jax import lax
from jax.experimental import pallas as pl
from jax.experimental.pallas import tpu as pltpu
from jax.experimental.pallas import tpu_sc as plsc

NUM_GRU = 127
OUT = 128
HEADS = 8
HID = 16
B = 2
N = 10000
E = 320000
NTOT = B * N
RW = 32
NTILES = 16
EDGE_BLK = 128
E_PER_TILE = 20096
E_PAD = E_PER_TILE * NTILES
NBLK = E_PER_TILE // EDGE_BLK
ROWW = 48
NPT = 1248
NCHUNK = 104


def _mm_body(x_ref, w_ref, o_ref):
    o_ref[...] = jnp.dot(x_ref[...], w_ref[0],
                         preferred_element_type=jnp.float32,
                         precision=lax.Precision.HIGHEST)


def _build_tables(x, wc):
    RB = 1000
    nrb = NTOT // RB
    return pl.pallas_call(
        _mm_body,
        grid=(8, nrb),
        in_specs=[
            pl.BlockSpec((RB, 128), lambda s, r: (r, 0)),
            pl.BlockSpec((1, 128, RW), lambda s, r: (s, 0, 0)),
        ],
        out_specs=pl.BlockSpec((RB, RW), lambda s, r: (s * nrb + r, 0)),
        out_shape=jax.ShapeDtypeStruct((8 * NTOT, RW), jnp.float32),
    )(x, wc)


def _div_body(a_ref, s_ref, o_ref):
    blk = a_ref[...]
    num = blk[:, :RW]
    den_rep = jnp.dot(blk, s_ref[...], preferred_element_type=jnp.float32,
                      precision=lax.Precision.HIGHEST)
    o_ref[...] = num / (den_rep + 1e-16)


def _divide(acc4, sel):
    RB = 2000
    nrb = 4 * NTOT // RB
    return pl.pallas_call(
        _div_body,
        grid=(nrb,),
        in_specs=[
            pl.BlockSpec((RB, ROWW), lambda i: (i, 0)),
            pl.BlockSpec((ROWW, RW), lambda i: (0, 0)),
        ],
        out_specs=pl.BlockSpec((RB, RW), lambda i: (i, 0)),
        out_shape=jax.ShapeDtypeStruct((4 * NTOT, RW), jnp.float32),
    )(acc4, sel)


def _sc_body(tab_hbm, tabidx_hbm, att_hbm, acc_hbm,
             att_v, idxb, srows4, drows4, stage2, zbuf, bounce,
             acc, sem_rows, sem_idx, sem_add):
    c = lax.axis_index("c")
    s = lax.axis_index("s")
    iota16 = lax.iota(jnp.int32, 16)
    zero16 = jnp.zeros((16,), jnp.float32)

    def zb(i, _):
        for j in range(ROWW // 16):
            zbuf[i, pl.ds(j * 16, 16)] = zero16
        return 0
    lax.fori_loop(0, NCHUNK, zb, 0)

    pltpu.sync_copy(att_hbm.at[c], att_v)
    n0 = s * NPT

    for r in range(2):
        for k in range(NPT // NCHUNK):
            pltpu.sync_copy(zbuf, acc.at[pl.ds(n0 + k * NCHUNK, NCHUNK)])

        @pl.when(s == 0)
        def _():
            pltpu.sync_copy(zbuf.at[pl.ds(0, 32)],
                            acc.at[pl.ds(NTILES * NPT, 32)])

        plsc.subcore_barrier()

        sec = c * 2 + r
        e_base = s * E_PER_TILE
        pltpu.sync_copy(tabidx_hbm.at[sec, :, pl.ds(e_base, EDGE_BLK)],
                        idxb.at[0])
        pltpu.sync_copy(
            tabidx_hbm.at[sec, :, pl.ds(e_base + EDGE_BLK, EDGE_BLK)],
            idxb.at[1])
        pltpu.async_copy(tab_hbm.at[idxb.at[0, 0]], srows4.at[0], sem_rows)
        pltpu.async_copy(tab_hbm.at[idxb.at[0, 1]], drows4.at[0], sem_rows)
        pltpu.async_copy(tab_hbm.at[idxb.at[1, 0]], srows4.at[1], sem_rows)
        pltpu.async_copy(tab_hbm.at[idxb.at[1, 1]], drows4.at[1], sem_rows)
        pltpu.async_copy(
            tabidx_hbm.at[sec, :, pl.ds(e_base + 2 * EDGE_BLK, EDGE_BLK)],
            idxb.at[2], sem_idx)

        def edge_block(b, _, r=r, sec=sec):
            p2 = lax.rem(b, 2)
            p4 = lax.rem(b, 4)
            e0 = e_base + b * EDGE_BLK
            pltpu.make_async_copy(
                tab_hbm.at[idxb.at[lax.rem(b, 8), 0]], srows4.at[p4],
                sem_rows).wait()
            pltpu.make_async_copy(
                tab_hbm.at[idxb.at[lax.rem(b, 8), 1]], drows4.at[p4],
                sem_rows).wait()

            @pl.when(b + 2 < NBLK)
            def _():
                slot2 = lax.rem(b + 2, 8)
                pltpu.make_async_copy(
                    tabidx_hbm.at[sec, :,
                                  pl.ds(e0 + 2 * EDGE_BLK, EDGE_BLK)],
                    idxb.at[slot2], sem_idx).wait()
                pltpu.async_copy(tab_hbm.at[idxb.at[slot2, 0]],
                                 srows4.at[lax.rem(b + 2, 4)], sem_rows)
                pltpu.async_copy(tab_hbm.at[idxb.at[slot2, 1]],
                                 drows4.at[lax.rem(b + 2, 4)], sem_rows)

            @pl.when(b >= 2)
            def _():
                pltpu.make_async_copy(
                    stage2.at[p2], acc.at[idxb.at[lax.rem(b + 6, 8), 2]],
                    sem_add).wait()

            @pl.when(b + 3 < NBLK)
            def _():
                pltpu.async_copy(
                    tabidx_hbm.at[sec, :,
                                  pl.ds(e0 + 3 * EDGE_BLK, EDGE_BLK)],
                    idxb.at[lax.rem(b + 3, 8)], sem_idx)

            @pl.when(b >= 1)
            def _():
                pltpu.async_copy(stage2.at[1 - p2],
                                 acc.at[idxb.at[lax.rem(b + 7, 8), 2]],
                                 sem_add, add=True)

            for h in range(2):
                attvecs = [att_v[(2 * r + h) * 16 + cc] for cc in range(16)]

                def grp(g, _, h=h, attvecs=attvecs, e0=e0, p2=p2, p4=p4):
                    rid = iota16 + g * 16
                    valid = (rid + e0) < E
                    pv2 = jnp.full((16,), 0, jnp.int32) + p2
                    pv4 = jnp.full((16,), 0, jnp.int32) + p4
                    logit = jnp.zeros((16,), jnp.float32)
                    svals = []
                    for cc in range(16):
                        colv = jnp.full((16,), h * 16 + cc, jnp.int32)
                        sv = plsc.load_gather(srows4, [pv4, rid, colv])
                        dv = plsc.load_gather(drows4, [pv4, rid, colv])
                        z = sv + dv
                        t = jnp.maximum(z, 0.2 * z)
                        logit = logit + t * attvecs[cc]
                        svals.append(sv)
                    ex = jnp.exp(logit)
                    ex = jnp.where(valid, ex, 0.0)
                    for cc in range(16):
                        colv = jnp.full((16,), h * 16 + cc, jnp.int32)
                        plsc.store_scatter(stage2, [pv2, rid, colv],
                                           svals[cc] * ex)
                    plsc.store_scatter(
                        stage2,
                        [pv2, rid, jnp.full((16,), RW + h, jnp.int32)],
                        ex)
                    return 0

                lax.fori_loop(0, EDGE_BLK // 16, grp, 0)
            return 0

        lax.fori_loop(0, NBLK, edge_block, 0)
        pltpu.make_async_copy(
            stage2.at[(NBLK - 2) % 2],
            acc.at[idxb.at[(NBLK - 2) % 8, 2]], sem_add).wait()
        pltpu.sync_copy(att_hbm.at[c], att_v)
        pltpu.sync_copy(stage2.at[(NBLK - 1) % 2],
                        acc.at[idxb.at[(NBLK - 1) % 8, 2]], add=True)
        plsc.subcore_barrier()

        for k in range(NPT // NCHUNK):
            pltpu.sync_copy(acc.at[pl.ds(n0 + k * NCHUNK, NCHUNK)], bounce)
            pltpu.sync_copy(bounce,
                            acc_hbm.at[c, r, pl.ds(n0 + k * NCHUNK, NCHUNK)])

        @pl.when(s == 0)
        def _():
            pltpu.sync_copy(acc.at[pl.ds(NTILES * NPT, 32)],
                            bounce.at[pl.ds(0, 32)])
            pltpu.sync_copy(bounce.at[pl.ds(0, 32)],
                            acc_hbm.at[c, r, pl.ds(NTILES * NPT, 32)])

        if r == 0:
            plsc.subcore_barrier()


_sc_kernel = functools.partial(
    pl.kernel,
    mesh=plsc.VectorSubcoreMesh(core_axis_name="c", subcore_axis_name="s"),
    out_type=jax.ShapeDtypeStruct((2, 2, NTOT, ROWW), jnp.float32),
    compiler_params=pltpu.CompilerParams(
        needs_layout_passes=False, use_tc_tiling_on_sc=False),
    scratch_types=[
        pltpu.VMEM((64, 16), jnp.float32),
        pltpu.VMEM((8, 3, EDGE_BLK), jnp.int32),
        pltpu.VMEM((4, EDGE_BLK, RW), jnp.float32),
        pltpu.VMEM((4, EDGE_BLK, RW), jnp.float32),
        pltpu.VMEM((2, EDGE_BLK, ROWW), jnp.float32),
        pltpu.VMEM((NCHUNK, ROWW), jnp.float32),
        pltpu.VMEM((NCHUNK, ROWW), jnp.float32),
        pltpu.VMEM_SHARED((NTOT, ROWW), jnp.float32),
        pltpu.SemaphoreType.DMA,
        pltpu.SemaphoreType.DMA,
        pltpu.SemaphoreType.DMA,
    ],
)(_sc_body)


def kernel(inputs, edge_index, edge_att, hidden_state, Wl, Wr, att, bias):
    x = jnp.concatenate(
        [inputs.reshape(B * N, 1), hidden_state.reshape(B * N, NUM_GRU)],
        axis=1)
    wc = jnp.stack([Wl[:, 0:32], Wl[:, 32:64], Wl[:, 64:96], Wl[:, 96:128],
                    Wr[:, 0:32], Wr[:, 32:64], Wr[:, 64:96], Wr[:, 96:128]],
                   axis=0)
    tab = _build_tables(x, wc)
    src = edge_index[0].astype(jnp.int32)
    dst = edge_index[1].astype(jnp.int32)
    pad = jnp.zeros((E_PAD - E,), jnp.int32)
    src_p = jnp.concatenate([src, pad])
    dst_p = jnp.concatenate([dst, pad])
    secs = jnp.arange(4, dtype=jnp.int32) * NTOT
    tabidx = jnp.stack(
        [jnp.broadcast_to(src_p, (4, E_PAD)) + secs[:, None],
         jnp.broadcast_to(dst_p, (4, E_PAD)) + (4 * NTOT + secs)[:, None],
         jnp.broadcast_to(dst_p, (4, E_PAD))], axis=1)
    att_exp = jnp.broadcast_to(
        att.astype(jnp.float32).reshape(2, 64)[:, :, None], (2, 64, 16))
    acc4 = _sc_kernel(tab, tabidx, att_exp)
    sel = jnp.zeros((ROWW, RW), jnp.float32).at[RW:RW + 2].set(
        jnp.repeat(jnp.eye(2, dtype=jnp.float32), HID, axis=1))
    d = _divide(acc4.reshape(4 * NTOT, ROWW), sel).reshape(4, NTOT, RW)
    full = jnp.concatenate([d[0], d[1], d[2], d[3]], axis=1)
    full = full + bias[None, :]
    return full.reshape(B, N * OUT)


# speedup vs baseline: 1.7274x; 1.7274x over previous
import functools

import jax
import jax.numpy as jnp
from jax import lax
from jax.experimental import pallas as pl
from jax.experimental.pallas import tpu as pltpu
from jax.experimental.pallas import tpu_sc as plsc

NUM_GRU = 127
OUT = 128
HEADS = 8
HID = 16
B = 2
N = 10000
E = 320000
NTOT = B * N
RW = 32
NTILES = 16
EDGE_BLK = 128
E_PER_TILE = 20096
E_PAD = E_PER_TILE * NTILES
NBLK = E_PER_TILE // EDGE_BLK
ROWW = 48
NPT = 1248
NCHUNK = 104


def _mm_body(x_ref, w_ref, o_ref):
    r = jnp.dot(x_ref[...], w_ref[0],
                preferred_element_type=jnp.float32,
                precision=lax.Precision.HIGHEST)
    o_ref[...] = r.astype(jnp.bfloat16)


def _build_tables(x, wc):
    RB = 1000
    nrb = NTOT // RB
    return pl.pallas_call(
        _mm_body,
        grid=(8, nrb),
        in_specs=[
            pl.BlockSpec((RB, 128), lambda s, r: (r, 0)),
            pl.BlockSpec((1, 128, RW), lambda s, r: (s, 0, 0)),
        ],
        out_specs=pl.BlockSpec((RB, RW), lambda s, r: (s * nrb + r, 0)),
        out_shape=jax.ShapeDtypeStruct((8 * NTOT, RW), jnp.bfloat16),
    )(x, wc)


def _div_body(a_ref, s_ref, o_ref):
    blk = a_ref[...]
    num = blk[:, :RW]
    den_rep = jnp.dot(blk, s_ref[...], preferred_element_type=jnp.float32,
                      precision=lax.Precision.HIGHEST)
    o_ref[...] = num / (den_rep + 1e-16)


def _divide(acc4, sel):
    RB = 2000
    nrb = 4 * NTOT // RB
    return pl.pallas_call(
        _div_body,
        grid=(nrb,),
        in_specs=[
            pl.BlockSpec((RB, ROWW), lambda i: (i, 0)),
            pl.BlockSpec((ROWW, RW), lambda i: (0, 0)),
        ],
        out_specs=pl.BlockSpec((RB, RW), lambda i: (i, 0)),
        out_shape=jax.ShapeDtypeStruct((4 * NTOT, RW), jnp.float32),
    )(acc4, sel)


def _sc_body(tab_hbm, tabidx_hbm, att_hbm, acc_hbm,
             att_v, idxb, srows4, drows4, stage2, zbuf, bounce,
             acc, sem_rows, sem_idx, sem_add):
    c = lax.axis_index("c")
    s = lax.axis_index("s")
    iota16 = lax.iota(jnp.int32, 16)
    zero16 = jnp.zeros((16,), jnp.float32)

    def zb(i, _):
        for j in range(ROWW // 16):
            zbuf[i, pl.ds(j * 16, 16)] = zero16
        return 0
    lax.fori_loop(0, NCHUNK, zb, 0)

    pltpu.sync_copy(att_hbm.at[c], att_v)
    n0 = s * NPT

    for r in range(2):
        for k in range(NPT // NCHUNK):
            pltpu.sync_copy(zbuf, acc.at[pl.ds(n0 + k * NCHUNK, NCHUNK)])

        @pl.when(s == 0)
        def _():
            pltpu.sync_copy(zbuf.at[pl.ds(0, 32)],
                            acc.at[pl.ds(NTILES * NPT, 32)])

        plsc.subcore_barrier()

        sec = c * 2 + r
        e_base = s * E_PER_TILE
        pltpu.sync_copy(tabidx_hbm.at[sec, :, pl.ds(e_base, EDGE_BLK)],
                        idxb.at[0])
        pltpu.sync_copy(
            tabidx_hbm.at[sec, :, pl.ds(e_base + EDGE_BLK, EDGE_BLK)],
            idxb.at[1])
        pltpu.async_copy(tab_hbm.at[idxb.at[0, 0]], srows4.at[0], sem_rows)
        pltpu.async_copy(tab_hbm.at[idxb.at[0, 1]], drows4.at[0], sem_rows)
        pltpu.async_copy(tab_hbm.at[idxb.at[1, 0]], srows4.at[1], sem_rows)
        pltpu.async_copy(tab_hbm.at[idxb.at[1, 1]], drows4.at[1], sem_rows)
        pltpu.async_copy(
            tabidx_hbm.at[sec, :, pl.ds(e_base + 2 * EDGE_BLK, EDGE_BLK)],
            idxb.at[2], sem_idx)

        def edge_block(b, _, r=r, sec=sec):
            p2 = lax.rem(b, 2)
            p4 = lax.rem(b, 4)
            e0 = e_base + b * EDGE_BLK
            pltpu.make_async_copy(
                tab_hbm.at[idxb.at[lax.rem(b, 8), 0]], srows4.at[p4],
                sem_rows).wait()
            pltpu.make_async_copy(
                tab_hbm.at[idxb.at[lax.rem(b, 8), 1]], drows4.at[p4],
                sem_rows).wait()

            @pl.when(b + 2 < NBLK)
            def _():
                slot2 = lax.rem(b + 2, 8)
                pltpu.make_async_copy(
                    tabidx_hbm.at[sec, :,
                                  pl.ds(e0 + 2 * EDGE_BLK, EDGE_BLK)],
                    idxb.at[slot2], sem_idx).wait()
                pltpu.async_copy(tab_hbm.at[idxb.at[slot2, 0]],
                                 srows4.at[lax.rem(b + 2, 4)], sem_rows)
                pltpu.async_copy(tab_hbm.at[idxb.at[slot2, 1]],
                                 drows4.at[lax.rem(b + 2, 4)], sem_rows)

            @pl.when(b >= 2)
            def _():
                pltpu.make_async_copy(
                    stage2.at[p2], acc.at[idxb.at[lax.rem(b + 6, 8), 2]],
                    sem_add).wait()

            @pl.when(b + 3 < NBLK)
            def _():
                pltpu.async_copy(
                    tabidx_hbm.at[sec, :,
                                  pl.ds(e0 + 3 * EDGE_BLK, EDGE_BLK)],
                    idxb.at[lax.rem(b + 3, 8)], sem_idx)

            @pl.when(b >= 1)
            def _():
                pltpu.async_copy(stage2.at[1 - p2],
                                 acc.at[idxb.at[lax.rem(b + 7, 8), 2]],
                                 sem_add, add=True)

            for h in range(2):
                attvecs = [att_v[(2 * r + h) * 16 + cc] for cc in range(16)]

                def grp(g, _, h=h, attvecs=attvecs, e0=e0, p2=p2, p4=p4):
                    rid = iota16 + g * 16
                    valid = (rid + e0) < E
                    pv2 = jnp.full((16,), 0, jnp.int32) + p2
                    pv4 = jnp.full((16,), 0, jnp.int32) + p4
                    logit = jnp.zeros((16,), jnp.float32)
                    svals = []
                    for cc2 in range(8):
                        colv = jnp.full((16,), h * 8 + cc2, jnp.int32)
                        sp = plsc.load_gather(srows4, [pv4, rid, colv])
                        dp = plsc.load_gather(drows4, [pv4, rid, colv])
                        sa, sb = plsc.unpack(
                            plsc.bitcast(sp, jnp.bfloat16),
                            format=plsc.PackFormat.INTERLEAVED,
                            preferred_element_type=jnp.float32)
                        da, db = plsc.unpack(
                            plsc.bitcast(dp, jnp.bfloat16),
                            format=plsc.PackFormat.INTERLEAVED,
                            preferred_element_type=jnp.float32)
                        for sv, dv, cc in ((sa, da, 2 * cc2),
                                           (sb, db, 2 * cc2 + 1)):
                            z = sv + dv
                            t = jnp.maximum(z, 0.2 * z)
                            logit = logit + t * attvecs[cc]
                            svals.append(sv)
                    ex = jnp.exp(logit)
                    ex = jnp.where(valid, ex, 0.0)
                    for cc in range(16):
                        colv = jnp.full((16,), h * 16 + cc, jnp.int32)
                        plsc.store_scatter(stage2, [pv2, rid, colv],
                                           svals[cc] * ex)
                    plsc.store_scatter(
                        stage2,
                        [pv2, rid, jnp.full((16,), RW + h, jnp.int32)],
                        ex)
                    return 0

                lax.fori_loop(0, EDGE_BLK // 16, grp, 0)
            return 0

        lax.fori_loop(0, NBLK, edge_block, 0)
        pltpu.make_async_copy(
            stage2.at[(NBLK - 2) % 2],
            acc.at[idxb.at[(NBLK - 2) % 8, 2]], sem_add).wait()
        pltpu.sync_copy(att_hbm.at[c], att_v)
        pltpu.sync_copy(stage2.at[(NBLK - 1) % 2],
                        acc.at[idxb.at[(NBLK - 1) % 8, 2]], add=True)
        plsc.subcore_barrier()

        for k in range(NPT // NCHUNK):
            pltpu.sync_copy(acc.at[pl.ds(n0 + k * NCHUNK, NCHUNK)], bounce)
            pltpu.sync_copy(bounce,
                            acc_hbm.at[c, r, pl.ds(n0 + k * NCHUNK, NCHUNK)])

        @pl.when(s == 0)
        def _():
            pltpu.sync_copy(acc.at[pl.ds(NTILES * NPT, 32)],
                            bounce.at[pl.ds(0, 32)])
            pltpu.sync_copy(bounce.at[pl.ds(0, 32)],
                            acc_hbm.at[c, r, pl.ds(NTILES * NPT, 32)])

        if r == 0:
            plsc.subcore_barrier()


_sc_kernel = functools.partial(
    pl.kernel,
    mesh=plsc.VectorSubcoreMesh(core_axis_name="c", subcore_axis_name="s"),
    out_type=jax.ShapeDtypeStruct((2, 2, NTOT, ROWW), jnp.float32),
    compiler_params=pltpu.CompilerParams(
        needs_layout_passes=False, use_tc_tiling_on_sc=False),
    scratch_types=[
        pltpu.VMEM((64, 16), jnp.float32),
        pltpu.VMEM((8, 3, EDGE_BLK), jnp.int32),
        pltpu.VMEM((4, EDGE_BLK, RW // 2), jnp.int32),
        pltpu.VMEM((4, EDGE_BLK, RW // 2), jnp.int32),
        pltpu.VMEM((2, EDGE_BLK, ROWW), jnp.float32),
        pltpu.VMEM((NCHUNK, ROWW), jnp.float32),
        pltpu.VMEM((NCHUNK, ROWW), jnp.float32),
        pltpu.VMEM_SHARED((NTOT, ROWW), jnp.float32),
        pltpu.SemaphoreType.DMA,
        pltpu.SemaphoreType.DMA,
        pltpu.SemaphoreType.DMA,
    ],
)(_sc_body)


def kernel(inputs, edge_index, edge_att, hidden_state, Wl, Wr, att, bias):
    x = jnp.concatenate(
        [inputs.reshape(B * N, 1), hidden_state.reshape(B * N, NUM_GRU)],
        axis=1)
    wc = jnp.stack([Wl[:, 0:32], Wl[:, 32:64], Wl[:, 64:96], Wl[:, 96:128],
                    Wr[:, 0:32], Wr[:, 32:64], Wr[:, 64:96], Wr[:, 96:128]],
                   axis=0)
    tab = lax.bitcast_convert_type(
        _build_tables(x, wc).reshape(8 * NTOT, RW // 2, 2), jnp.int32)
    src = edge_index[0].astype(jnp.int32)
    dst = edge_index[1].astype(jnp.int32)
    pad = jnp.zeros((E_PAD - E,), jnp.int32)
    src_p = jnp.concatenate([src, pad])
    dst_p = jnp.concatenate([dst, pad])
    secs = jnp.arange(4, dtype=jnp.int32) * NTOT
    tabidx = jnp.stack(
        [jnp.broadcast_to(src_p, (4, E_PAD)) + secs[:, None],
         jnp.broadcast_to(dst_p, (4, E_PAD)) + (4 * NTOT + secs)[:, None],
         jnp.broadcast_to(dst_p, (4, E_PAD))], axis=1)
    att_exp = jnp.broadcast_to(
        att.astype(jnp.float32).reshape(2, 64)[:, :, None], (2, 64, 16))
    acc4 = _sc_kernel(tab, tabidx, att_exp)
    sel = jnp.zeros((ROWW, RW), jnp.float32).at[RW:RW + 2].set(
        jnp.repeat(jnp.eye(2, dtype=jnp.float32), HID, axis=1))
    d = _divide(acc4.reshape(4 * NTOT, ROWW), sel).reshape(4, NTOT, RW)
    full = jnp.concatenate([d[0], d[1], d[2], d[3]], axis=1)
    full = full + bias[None, :]
    return full.reshape(B, N * OUT)
